# baseline (speedup 1.0000x reference)
import functools

import jax
import jax.numpy as jnp
from jax import lax
from jax.experimental import pallas as pl
from jax.experimental.pallas import tpu as pltpu
from jax.experimental.pallas import tpu_sc as plsc

N = 10000
E = 640000
IN_DIM = 128
MD = 32
MD4 = 8

NC = 2
NS = 16
NW = NC * NS
EPW = E // NW
CE1 = 40
CE2 = 80
RPT = 640
RPT_LAST = N - 15 * RPT
ZR = 40

ROW1 = 144
ROW2 = 48

_F32 = jnp.float32
_BLK = 2000
_GRID = N // _BLK


def _mm(x, w, b):
    return jnp.dot(x, w, preferred_element_type=_F32) + b


def _row_spec(cols):
    return pl.BlockSpec((_BLK, cols), lambda i: (i, 0))


def _full_spec(shape):
    return pl.BlockSpec(shape, lambda i: tuple(0 for _ in shape))


def _tca_body(nf, ed, wn, bn, wq, bq, wk, bk, wv, bv, wsk, bsk,
              q1, kv1, skip1, edi):
    ne = _mm(nf[...], wn[...], bn[...])
    q1[...] = _mm(ne, wq[...], bq[...])
    k1 = _mm(ne, wk[...], bk[...])
    v1 = _mm(ne, wv[...], bv[...])
    kv1[...] = jnp.concatenate([k1, v1], axis=1)
    skip1[...] = _mm(ne, wsk[...], bsk[...])
    edi[...] = jnp.clip(ed[...].astype(jnp.int32), 0, N - 1)


def _tcb_body(pa, pb, skip1, wq, bq, wk, bk, wv, bv, wsk, bsk,
              q2, kv2, skip2):
    num = pa[...][:, :128] + pb[...][:, :128]
    den = pa[...][:, 128:132] + pb[...][:, 128:132]
    acc = jnp.zeros((_BLK, MD), _F32)
    for h in range(4):
        nh = num[:, 32 * h:32 * h + 32]
        dh = jnp.broadcast_to(den[:, h:h + 1], (_BLK, MD))
        acc = acc + nh / (dh + 1e-16)
    x1 = jnp.maximum(0.25 * acc + skip1[...], 0.0)
    q2[...] = _mm(x1, wq[...], bq[...])
    k2 = _mm(x1, wk[...], bk[...])
    v2 = _mm(x1, wv[...], bv[...])
    kv2[...] = jnp.concatenate([k2, v2], axis=1)
    skip2[...] = _mm(x1, wsk[...], bsk[...])


def _tcc_body(pa, pb, skip2, wo, bo, out):
    num = pa[...][:, :32] + pb[...][:, :32]
    den = jnp.broadcast_to(pa[...][:, 32:33] + pb[...][:, 32:33], (_BLK, MD))
    x2 = jnp.maximum(num / (den + 1e-16) + skip2[...], 0.0)
    out[...] = _mm(x2, wo[...], bo[...])


_MESH = plsc.VectorSubcoreMesh(core_axis_name="c", subcore_axis_name="s")
_SC_PARAMS = pltpu.CompilerParams(use_tc_tiling_on_sc=False,
                                  needs_layout_passes=False)


def _sc_edge_pass(nheads, qdim, row_words, ce,
                  src_h, dst_h, q_h, kv_h, out_h,
                  src_c0, src_c1, dst_c0, dst_c1, dst_s0, dst_s1,
                  qv0, qv1, kv0, kv1, con_v, acc_sh,
                  si0, si1, sq0, sq1, skv0, skv1, ss):
    cid = lax.axis_index("c")
    sid = lax.axis_index("s")
    wid = cid * NS + sid
    nchunk = EPW // ce

    src_c = (src_c0, src_c1)
    dst_c = (dst_c0, dst_c1)
    dst_s = (dst_s0, dst_s1)
    qv = (qv0, qv1)
    kv = (kv0, kv1)
    si = (si0, si1)
    sq = (sq0, sq1)
    skv = (skv0, skv1)

    iota16 = lax.iota(jnp.int32, 16)
    z16 = jnp.zeros((16,), _F32)
    nb = row_words // 16

    def zrow(r, c):
        for i in range(nb):
            con_v[r, pl.ds(16 * i, 16)] = z16
        return c
    lax.fori_loop(0, ce, zrow, 0)

    row0 = sid * RPT
    ncopy = jnp.where(sid < 15, RPT // ce, RPT_LAST // ce)

    def init(i, c):
        pltpu.sync_copy(con_v, acc_sh.at[pl.ds(row0 + i * ce, ce)])
        return c
    lax.fori_loop(0, ncopy, init, 0)
    plsc.subcore_barrier()

    ebase = wid * EPW

    def fetch_idx(chunk_id, p):
        b = pl.multiple_of(ebase + chunk_id * ce, 8)
        pltpu.async_copy(src_h.at[pl.ds(b, ce)], src_c[p], si[p])
        pltpu.async_copy(dst_h.at[pl.ds(b, ce)], dst_c[p], si[p])

    def wait_idx(p):
        b0 = pl.multiple_of(ebase, 8)
        pltpu.make_async_copy(src_h.at[pl.ds(b0, ce)], src_c[p], si[p]).wait()
        pltpu.make_async_copy(dst_h.at[pl.ds(b0, ce)], dst_c[p], si[p]).wait()

    def issue_gathers(p):
        pltpu.async_copy(q_h.at[dst_c[p]], qv[p], sq[p])
        pltpu.async_copy(kv_h.at[src_c[p]], kv[p], skv[p])

    def wait_gathers(p):
        pltpu.make_async_copy(q_h.at[dst_c[p]], qv[p], sq[p]).wait()
        pltpu.make_async_copy(kv_h.at[src_c[p]], kv[p], skv[p]).wait()

    def wait_scatter(p):
        pltpu.make_async_copy(con_v, acc_sh.at[dst_s[p]], ss).wait()

    def compute_chunk(p):
        qr, kr = qv[p], kv[p]

        @plsc.parallel_loop(0, ce, 1, unroll=8)
        def edge(e):
            dv = jnp.zeros((16,), _F32)
            for h in range(nheads):
                qa = qr[e, pl.ds(32 * h, 16)]
                qb = qr[e, pl.ds(32 * h + 16, 16)]
                ka = kr[e, pl.ds(32 * h, 16)]
                kb = kr[e, pl.ds(32 * h + 16, 16)]
                alpha = jnp.sum(qa * ka + qb * kb)
                ex = jnp.exp(jnp.full((16,), alpha, _F32))
                va = kr[e, pl.ds(qdim + 32 * h, 16)]
                vb = kr[e, pl.ds(qdim + 32 * h + 16, 16)]
                con_v[e, pl.ds(32 * h, 16)] = ex * va
                con_v[e, pl.ds(32 * h + 16, 16)] = ex * vb
                dv = jnp.where(iota16 == h, ex, dv)
            con_v[e, pl.ds(qdim, 16)] = dv

    def copy_dst(p):
        off = 0
        while off + 16 <= ce:
            dst_s[p][pl.ds(off, 16)] = dst_c[p][pl.ds(off, 16)]
            off += 16
        if off < ce:
            dst_s[p][pl.ds(ce - 16, 16)] = dst_c[p][pl.ds(ce - 16, 16)]

    fetch_idx(0, 0)
    wait_idx(0)
    issue_gathers(0)
    fetch_idx(1, 1)

    def pair(t, c):
        for p in (0, 1):
            jj = 2 * t + p
            q = 1 - p
            wait_gathers(p)
            wait_idx(q)
            issue_gathers(q)
            copy_dst(p)
            nxt = jnp.minimum(jj + 2, nchunk - 1)
            fetch_idx(nxt, p)
            if p == 0:
                @pl.when(t > 0)
                def _():
                    wait_scatter(p)
            else:
                wait_scatter(p)
            compute_chunk(p)
            pltpu.async_copy(con_v, acc_sh.at[dst_s[p]], ss, add=True)
        return c

    lax.fori_loop(0, nchunk // 2, pair, 0)

    wait_scatter(0)
    wait_gathers(0)
    wait_idx(1)
    plsc.subcore_barrier()

    @pl.when(sid < 15)
    def _():
        pltpu.sync_copy(acc_sh.at[pl.ds(row0, RPT)],
                        out_h.at[cid, pl.ds(row0, RPT)])

    @pl.when(sid == 15)
    def _():
        pltpu.sync_copy(acc_sh.at[pl.ds(15 * RPT, RPT_LAST)],
                        out_h.at[cid, pl.ds(15 * RPT, RPT_LAST)])


@functools.partial(
    pl.kernel,
    out_type=jax.ShapeDtypeStruct((NC, N, ROW1), _F32),
    mesh=_MESH,
    scratch_types=[
        pltpu.VMEM((CE1,), jnp.int32), pltpu.VMEM((CE1,), jnp.int32),
        pltpu.VMEM((CE1,), jnp.int32), pltpu.VMEM((CE1,), jnp.int32),
        pltpu.VMEM((CE1,), jnp.int32), pltpu.VMEM((CE1,), jnp.int32),
        pltpu.VMEM((CE1, 4 * MD), _F32), pltpu.VMEM((CE1, 4 * MD), _F32),
        pltpu.VMEM((CE1, 8 * MD), _F32), pltpu.VMEM((CE1, 8 * MD), _F32),
        pltpu.VMEM((CE1, ROW1), _F32),
        pltpu.VMEM_SHARED((N, ROW1), _F32),
        pltpu.SemaphoreType.DMA, pltpu.SemaphoreType.DMA,
        pltpu.SemaphoreType.DMA, pltpu.SemaphoreType.DMA,
        pltpu.SemaphoreType.DMA, pltpu.SemaphoreType.DMA,
        pltpu.SemaphoreType.DMA,
    ],
    compiler_params=_SC_PARAMS,
)
def _sc_layer1(*refs):
    _sc_edge_pass(4, 4 * MD, ROW1, CE1, *refs)


@functools.partial(
    pl.kernel,
    out_type=jax.ShapeDtypeStruct((NC, N, ROW2), _F32),
    mesh=_MESH,
    scratch_types=[
        pltpu.VMEM((CE2,), jnp.int32), pltpu.VMEM((CE2,), jnp.int32),
        pltpu.VMEM((CE2,), jnp.int32), pltpu.VMEM((CE2,), jnp.int32),
        pltpu.VMEM((CE2,), jnp.int32), pltpu.VMEM((CE2,), jnp.int32),
        pltpu.VMEM((CE2, MD), _F32), pltpu.VMEM((CE2, MD), _F32),
        pltpu.VMEM((CE2, 2 * MD), _F32), pltpu.VMEM((CE2, 2 * MD), _F32),
        pltpu.VMEM((CE2, ROW2), _F32),
        pltpu.VMEM_SHARED((N, ROW2), _F32),
        pltpu.SemaphoreType.DMA, pltpu.SemaphoreType.DMA,
        pltpu.SemaphoreType.DMA, pltpu.SemaphoreType.DMA,
        pltpu.SemaphoreType.DMA, pltpu.SemaphoreType.DMA,
        pltpu.SemaphoreType.DMA,
    ],
    compiler_params=_SC_PARAMS,
)
def _sc_layer2(*refs):
    _sc_edge_pass(1, MD, ROW2, CE2, *refs)


def kernel(edge_df, node_features, edge_mask, node_mask, Wt, bt, Wn, bn,
           Wq1, bq1, Wk1, bk1, Wv1, bv1, Wsk1, bsk1,
           Wq2, bq2, Wk2, bk2, Wv2, bv2, Wsk2, bsk2,
           Wih, bih, Whh, bhh, Wout, bout, node_memory):
    inv = 1.0 / jnp.sqrt(jnp.asarray(MD, _F32))
    te = Wt[:, 0] + bt

    def fold(W, b, scale=None):
        wa = W[:, :MD].T
        be = b + W[:, MD:] @ te
        if scale is not None:
            wa = wa * scale
            be = be * scale
        return wa, be.reshape(1, -1)

    wq1, bq1e = fold(Wq1, bq1, inv)
    wk1, bk1e = fold(Wk1, bk1)
    wv1, bv1e = fold(Wv1, bv1)
    wsk1, bsk1e = fold(Wsk1, bsk1)

    ed_view = edge_df.reshape(N, IN_DIM)

    q1, kv1, skip1, edi = pl.pallas_call(
        _tca_body,
        grid=(_GRID,),
        in_specs=[
            _row_spec(IN_DIM), _row_spec(IN_DIM),
            _full_spec((IN_DIM, MD)), _full_spec((1, MD)),
            _full_spec((MD, 4 * MD)), _full_spec((1, 4 * MD)),
            _full_spec((MD, 4 * MD)), _full_spec((1, 4 * MD)),
            _full_spec((MD, 4 * MD)), _full_spec((1, 4 * MD)),
            _full_spec((MD, MD)), _full_spec((1, MD)),
        ],
        out_specs=(
            _row_spec(4 * MD), _row_spec(8 * MD), _row_spec(MD),
            _row_spec(IN_DIM),
        ),
        out_shape=(
            jax.ShapeDtypeStruct((N, 4 * MD), _F32),
            jax.ShapeDtypeStruct((N, 8 * MD), _F32),
            jax.ShapeDtypeStruct((N, MD), _F32),
            jax.ShapeDtypeStruct((N, IN_DIM), jnp.int32),
        ),
    )(node_features, ed_view, Wn.T, bn.reshape(1, -1),
      wq1, bq1e, wk1, bk1e, wv1, bv1e, wsk1, bsk1e)

    sd = edi.reshape(E, 2)
    src = sd[:, 0]
    dst = sd[:, 1]

    part1 = _sc_layer1(src, dst, q1, kv1)

    q2, kv2, skip2 = pl.pallas_call(
        _tcb_body,
        grid=(_GRID,),
        in_specs=[
            _row_spec(ROW1), _row_spec(ROW1), _row_spec(MD),
            _full_spec((MD, MD)), _full_spec((1, MD)),
            _full_spec((MD, MD)), _full_spec((1, MD)),
            _full_spec((MD, MD)), _full_spec((1, MD)),
            _full_spec((MD, MD)), _full_spec((1, MD)),
        ],
        out_specs=(_row_spec(MD), _row_spec(2 * MD), _row_spec(MD)),
        out_shape=(
            jax.ShapeDtypeStruct((N, MD), _F32),
            jax.ShapeDtypeStruct((N, 2 * MD), _F32),
            jax.ShapeDtypeStruct((N, MD), _F32),
        ),
    )(part1[0], part1[1], skip1,
      Wq2.T * inv, (bq2 * inv).reshape(1, -1),
      Wk2.T, bk2.reshape(1, -1), Wv2.T, bv2.reshape(1, -1),
      Wsk2.T, bsk2.reshape(1, -1))

    part2 = _sc_layer2(src, dst, q2, kv2)

    output = pl.pallas_call(
        _tcc_body,
        grid=(_GRID,),
        in_specs=[
            _row_spec(ROW2), _row_spec(ROW2), _row_spec(MD),
            _full_spec((MD, MD)), _full_spec((1, MD)),
        ],
        out_specs=_row_spec(MD),
        out_shape=jax.ShapeDtypeStruct((N, MD), _F32),
    )(part2[0], part2[1], skip2, Wout.T, bout.reshape(1, -1))

    return output


# speedup vs baseline: 1.4955x; 1.4955x over previous
import functools

import jax
import jax.numpy as jnp
from jax import lax
from jax.experimental import pallas as pl
from jax.experimental.pallas import tpu as pltpu
from jax.experimental.pallas import tpu_sc as plsc

N = 10000
E = 640000
IN_DIM = 128
MD = 32
MD4 = 8

NC = 2
NS = 16
NW = NC * NS
EPW = E // NW
CE1 = 40
CE2 = 80
RPT = 640
RPT_LAST = N - 15 * RPT
ZR = 40

ROW1 = 144
ROW2 = 48

_F32 = jnp.float32
_BLK = 2000
_GRID = N // _BLK


def _mm(x, w, b):
    return jnp.dot(x, w, preferred_element_type=_F32) + b


def _row_spec(cols):
    return pl.BlockSpec((_BLK, cols), lambda i: (i, 0))


def _full_spec(shape):
    return pl.BlockSpec(shape, lambda i: tuple(0 for _ in shape))


def _tca_body(nf, ed, wn, bn, wq, bq, wk, bk, wv, bv, wsk, bsk,
              q1, kv1, skip1, edi):
    ne = _mm(nf[...], wn[...], bn[...])
    q1[...] = _mm(ne, wq[...], bq[...])
    k1 = _mm(ne, wk[...], bk[...])
    v1 = _mm(ne, wv[...], bv[...])
    kv1[...] = jnp.concatenate([k1, v1], axis=1)
    skip1[...] = _mm(ne, wsk[...], bsk[...])
    edi[...] = jnp.clip(ed[...].astype(jnp.int32), 0, N - 1)


def _tcb_body(pa, pb, skip1, wq, bq, wk, bk, wv, bv, wsk, bsk,
              q2, kv2, skip2):
    num = pa[...][:, :128] + pb[...][:, :128]
    den = pa[...][:, 128:132] + pb[...][:, 128:132]
    acc = jnp.zeros((_BLK, MD), _F32)
    for h in range(4):
        nh = num[:, 32 * h:32 * h + 32]
        dh = jnp.broadcast_to(den[:, h:h + 1], (_BLK, MD))
        acc = acc + nh / (dh + 1e-16)
    x1 = jnp.maximum(0.25 * acc + skip1[...], 0.0)
    q2[...] = _mm(x1, wq[...], bq[...])
    k2 = _mm(x1, wk[...], bk[...])
    v2 = _mm(x1, wv[...], bv[...])
    kv2[...] = jnp.concatenate([k2, v2], axis=1)
    skip2[...] = _mm(x1, wsk[...], bsk[...])


def _tcc_body(pa, pb, skip2, wo, bo, out):
    num = pa[...][:, :32] + pb[...][:, :32]
    den = jnp.broadcast_to(pa[...][:, 32:33] + pb[...][:, 32:33], (_BLK, MD))
    x2 = jnp.maximum(num / (den + 1e-16) + skip2[...], 0.0)
    out[...] = _mm(x2, wo[...], bo[...])


_MESH = plsc.VectorSubcoreMesh(core_axis_name="c", subcore_axis_name="s")
_SC_PARAMS = pltpu.CompilerParams(use_tc_tiling_on_sc=False,
                                  needs_layout_passes=False)


def _sc_edge_pass(nheads, qdim, row_words, ce,
                  src_h, dst_h, q_h, kv_h, out_h,
                  src_c0, src_c1, dst_c0, dst_c1, dst_s0, dst_s1,
                  qv0, qv1, kv0, kv1, con_v, acc_sh,
                  si0, si1, sq0, sq1, skv0, skv1, ss):
    cid = lax.axis_index("c")
    sid = lax.axis_index("s")
    wid = cid * NS + sid
    nchunk = EPW // ce

    src_c = (src_c0, src_c1)
    dst_c = (dst_c0, dst_c1)
    dst_s = (dst_s0, dst_s1)
    qv = (qv0, qv1)
    kv = (kv0, kv1)
    si = (si0, si1)
    sq = (sq0, sq1)
    skv = (skv0, skv1)

    iota16 = lax.iota(jnp.int32, 16)
    z16 = jnp.zeros((16,), _F32)
    nb = row_words // 16

    def zrow(r, c):
        for i in range(nb):
            con_v[r, pl.ds(16 * i, 16)] = z16
        return c
    lax.fori_loop(0, ce, zrow, 0)

    row0 = sid * RPT
    ncopy = jnp.where(sid < 15, RPT // ce, RPT_LAST // ce)

    def init(i, c):
        pltpu.sync_copy(con_v, acc_sh.at[pl.ds(row0 + i * ce, ce)])
        return c
    lax.fori_loop(0, ncopy, init, 0)
    plsc.subcore_barrier()

    ebase = wid * EPW

    def fetch_idx(chunk_id, p):
        b = pl.multiple_of(ebase + chunk_id * ce, 8)
        pltpu.async_copy(src_h.at[pl.ds(b, ce)], src_c[p], si[p])
        pltpu.async_copy(dst_h.at[pl.ds(b, ce)], dst_c[p], si[p])

    def wait_idx(p):
        b0 = pl.multiple_of(ebase, 8)
        pltpu.make_async_copy(src_h.at[pl.ds(b0, ce)], src_c[p], si[p]).wait()
        pltpu.make_async_copy(dst_h.at[pl.ds(b0, ce)], dst_c[p], si[p]).wait()

    def issue_gathers(p):
        pltpu.async_copy(q_h.at[dst_c[p]], qv[p], sq[p])
        pltpu.async_copy(kv_h.at[src_c[p]], kv[p], skv[p])

    def wait_gathers(p):
        pltpu.make_async_copy(q_h.at[dst_c[p]], qv[p], sq[p]).wait()
        pltpu.make_async_copy(kv_h.at[src_c[p]], kv[p], skv[p]).wait()

    def wait_scatter(p):
        pltpu.make_async_copy(con_v, acc_sh.at[dst_s[p]], ss).wait()

    def compute_chunk(p):
        qr, kr = qv[p], kv[p]

        @plsc.parallel_loop(0, ce, 1, unroll=4)
        def edge(e):
            dv = jnp.zeros((16,), _F32)
            for h in range(nheads):
                qa = qr[e, pl.ds(32 * h, 16)]
                qb = qr[e, pl.ds(32 * h + 16, 16)]
                ka = kr[e, pl.ds(32 * h, 16)]
                kb = kr[e, pl.ds(32 * h + 16, 16)]
                alpha = jnp.sum(qa * ka + qb * kb)
                ex = jnp.exp(jnp.full((16,), alpha, _F32))
                va = kr[e, pl.ds(qdim + 32 * h, 16)]
                vb = kr[e, pl.ds(qdim + 32 * h + 16, 16)]
                con_v[e, pl.ds(32 * h, 16)] = ex * va
                con_v[e, pl.ds(32 * h + 16, 16)] = ex * vb
                dv = jnp.where(iota16 == h, ex, dv)
            con_v[e, pl.ds(qdim, 16)] = dv

    def copy_dst(p):
        off = 0
        while off + 16 <= ce:
            dst_s[p][pl.ds(off, 16)] = dst_c[p][pl.ds(off, 16)]
            off += 16
        if off < ce:
            dst_s[p][pl.ds(ce - 16, 16)] = dst_c[p][pl.ds(ce - 16, 16)]

    fetch_idx(0, 0)
    wait_idx(0)
    issue_gathers(0)
    fetch_idx(1, 1)

    def pair(t, c):
        for p in (0, 1):
            jj = 2 * t + p
            q = 1 - p
            wait_gathers(p)
            wait_idx(q)
            issue_gathers(q)
            copy_dst(p)
            nxt = jnp.minimum(jj + 2, nchunk - 1)
            fetch_idx(nxt, p)
            if p == 0:
                @pl.when(t > 0)
                def _():
                    wait_scatter(p)
            else:
                wait_scatter(p)
            compute_chunk(p)
            pltpu.async_copy(con_v, acc_sh.at[dst_s[p]], ss, add=True)
        return c

    lax.fori_loop(0, nchunk // 2, pair, 0)

    wait_scatter(0)
    wait_gathers(0)
    wait_idx(1)
    plsc.subcore_barrier()

    @pl.when(sid < 15)
    def _():
        pltpu.sync_copy(acc_sh.at[pl.ds(row0, RPT)],
                        out_h.at[cid, pl.ds(row0, RPT)])

    @pl.when(sid == 15)
    def _():
        pltpu.sync_copy(acc_sh.at[pl.ds(15 * RPT, RPT_LAST)],
                        out_h.at[cid, pl.ds(15 * RPT, RPT_LAST)])


@functools.partial(
    pl.kernel,
    out_type=jax.ShapeDtypeStruct((NC, N, ROW1), _F32),
    mesh=_MESH,
    scratch_types=[
        pltpu.VMEM((CE1,), jnp.int32), pltpu.VMEM((CE1,), jnp.int32),
        pltpu.VMEM((CE1,), jnp.int32), pltpu.VMEM((CE1,), jnp.int32),
        pltpu.VMEM((CE1,), jnp.int32), pltpu.VMEM((CE1,), jnp.int32),
        pltpu.VMEM((CE1, 4 * MD), _F32), pltpu.VMEM((CE1, 4 * MD), _F32),
        pltpu.VMEM((CE1, 8 * MD), _F32), pltpu.VMEM((CE1, 8 * MD), _F32),
        pltpu.VMEM((CE1, ROW1), _F32),
        pltpu.VMEM_SHARED((N, ROW1), _F32),
        pltpu.SemaphoreType.DMA, pltpu.SemaphoreType.DMA,
        pltpu.SemaphoreType.DMA, pltpu.SemaphoreType.DMA,
        pltpu.SemaphoreType.DMA, pltpu.SemaphoreType.DMA,
        pltpu.SemaphoreType.DMA,
    ],
    compiler_params=_SC_PARAMS,
)
def _sc_layer1(*refs):
    _sc_edge_pass(4, 4 * MD, ROW1, CE1, *refs)


@functools.partial(
    pl.kernel,
    out_type=jax.ShapeDtypeStruct((NC, N, ROW2), _F32),
    mesh=_MESH,
    scratch_types=[
        pltpu.VMEM((CE2,), jnp.int32), pltpu.VMEM((CE2,), jnp.int32),
        pltpu.VMEM((CE2,), jnp.int32), pltpu.VMEM((CE2,), jnp.int32),
        pltpu.VMEM((CE2,), jnp.int32), pltpu.VMEM((CE2,), jnp.int32),
        pltpu.VMEM((CE2, MD), _F32), pltpu.VMEM((CE2, MD), _F32),
        pltpu.VMEM((CE2, 2 * MD), _F32), pltpu.VMEM((CE2, 2 * MD), _F32),
        pltpu.VMEM((CE2, ROW2), _F32),
        pltpu.VMEM_SHARED((N, ROW2), _F32),
        pltpu.SemaphoreType.DMA, pltpu.SemaphoreType.DMA,
        pltpu.SemaphoreType.DMA, pltpu.SemaphoreType.DMA,
        pltpu.SemaphoreType.DMA, pltpu.SemaphoreType.DMA,
        pltpu.SemaphoreType.DMA,
    ],
    compiler_params=_SC_PARAMS,
)
def _sc_layer2(*refs):
    _sc_edge_pass(1, MD, ROW2, CE2, *refs)


def kernel(edge_df, node_features, edge_mask, node_mask, Wt, bt, Wn, bn,
           Wq1, bq1, Wk1, bk1, Wv1, bv1, Wsk1, bsk1,
           Wq2, bq2, Wk2, bk2, Wv2, bv2, Wsk2, bsk2,
           Wih, bih, Whh, bhh, Wout, bout, node_memory):
    inv = 1.0 / jnp.sqrt(jnp.asarray(MD, _F32))
    te = Wt[:, 0] + bt

    def fold(W, b, scale=None):
        wa = W[:, :MD].T
        be = b + W[:, MD:] @ te
        if scale is not None:
            wa = wa * scale
            be = be * scale
        return wa, be.reshape(1, -1)

    wq1, bq1e = fold(Wq1, bq1, inv)
    wk1, bk1e = fold(Wk1, bk1)
    wv1, bv1e = fold(Wv1, bv1)
    wsk1, bsk1e = fold(Wsk1, bsk1)

    ed_view = edge_df.reshape(N, IN_DIM)

    q1, kv1, skip1, edi = pl.pallas_call(
        _tca_body,
        grid=(_GRID,),
        in_specs=[
            _row_spec(IN_DIM), _row_spec(IN_DIM),
            _full_spec((IN_DIM, MD)), _full_spec((1, MD)),
            _full_spec((MD, 4 * MD)), _full_spec((1, 4 * MD)),
            _full_spec((MD, 4 * MD)), _full_spec((1, 4 * MD)),
            _full_spec((MD, 4 * MD)), _full_spec((1, 4 * MD)),
            _full_spec((MD, MD)), _full_spec((1, MD)),
        ],
        out_specs=(
            _row_spec(4 * MD), _row_spec(8 * MD), _row_spec(MD),
            _row_spec(IN_DIM),
        ),
        out_shape=(
            jax.ShapeDtypeStruct((N, 4 * MD), _F32),
            jax.ShapeDtypeStruct((N, 8 * MD), _F32),
            jax.ShapeDtypeStruct((N, MD), _F32),
            jax.ShapeDtypeStruct((N, IN_DIM), jnp.int32),
        ),
    )(node_features, ed_view, Wn.T, bn.reshape(1, -1),
      wq1, bq1e, wk1, bk1e, wv1, bv1e, wsk1, bsk1e)

    sd = edi.reshape(E, 2)
    src = sd[:, 0]
    dst = sd[:, 1]

    part1 = _sc_layer1(src, dst, q1, kv1)

    q2, kv2, skip2 = pl.pallas_call(
        _tcb_body,
        grid=(_GRID,),
        in_specs=[
            _row_spec(ROW1), _row_spec(ROW1), _row_spec(MD),
            _full_spec((MD, MD)), _full_spec((1, MD)),
            _full_spec((MD, MD)), _full_spec((1, MD)),
            _full_spec((MD, MD)), _full_spec((1, MD)),
            _full_spec((MD, MD)), _full_spec((1, MD)),
        ],
        out_specs=(_row_spec(MD), _row_spec(2 * MD), _row_spec(MD)),
        out_shape=(
            jax.ShapeDtypeStruct((N, MD), _F32),
            jax.ShapeDtypeStruct((N, 2 * MD), _F32),
            jax.ShapeDtypeStruct((N, MD), _F32),
        ),
    )(part1[0], part1[1], skip1,
      Wq2.T * inv, (bq2 * inv).reshape(1, -1),
      Wk2.T, bk2.reshape(1, -1), Wv2.T, bv2.reshape(1, -1),
      Wsk2.T, bsk2.reshape(1, -1))

    part2 = _sc_layer2(src, dst, q2, kv2)

    output = pl.pallas_call(
        _tcc_body,
        grid=(_GRID,),
        in_specs=[
            _row_spec(ROW2), _row_spec(ROW2), _row_spec(MD),
            _full_spec((MD, MD)), _full_spec((1, MD)),
        ],
        out_specs=_row_spec(MD),
        out_shape=jax.ShapeDtypeStruct((N, MD), _F32),
    )(part2[0], part2[1], skip2, Wout.T, bout.reshape(1, -1))

    return output


# speedup vs baseline: 2.1216x; 1.4187x over previous
import functools

import jax
import jax.numpy as jnp
from jax import lax
from jax.experimental import pallas as pl
from jax.experimental.pallas import tpu as pltpu
from jax.experimental.pallas import tpu_sc as plsc

N = 10000
E = 640000
IN_DIM = 128
MD = 32
MD4 = 8

NC = 2
NS = 16
NW = NC * NS
EPW = E // NW
CE1 = 40
CE2 = 80
RPT = 640
RPT_LAST = N - 15 * RPT
ZR = 40

ROW1 = 144
ROW2 = 48

_F32 = jnp.float32
_BLK = 2000
_GRID = N // _BLK


def _mm(x, w, b):
    return jnp.dot(x, w, preferred_element_type=_F32) + b


def _row_spec(cols):
    return pl.BlockSpec((_BLK, cols), lambda i: (i, 0))


def _full_spec(shape):
    return pl.BlockSpec(shape, lambda i: tuple(0 for _ in shape))


def _tca_body(nf, ed, wn, bn, wq, bq, wk, bk, wv, bv, wsk, bsk,
              q1, kv1, skip1, edi):
    ne = _mm(nf[...], wn[...], bn[...])
    q1[...] = _mm(ne, wq[...], bq[...])
    k1 = _mm(ne, wk[...], bk[...])
    v1 = _mm(ne, wv[...], bv[...])
    kv1[...] = jnp.concatenate([k1, v1], axis=1)
    skip1[...] = _mm(ne, wsk[...], bsk[...])
    edi[...] = jnp.clip(ed[...].astype(jnp.int32), 0, N - 1)


def _tcb_body(pa, pb, skip1, wq, bq, wk, bk, wv, bv, wsk, bsk,
              q2, kv2, skip2):
    num = pa[...][:, :128] + pb[...][:, :128]
    den = pa[...][:, 128:132] + pb[...][:, 128:132]
    acc = jnp.zeros((_BLK, MD), _F32)
    for h in range(4):
        nh = num[:, 32 * h:32 * h + 32]
        dh = jnp.broadcast_to(den[:, h:h + 1], (_BLK, MD))
        acc = acc + nh / (dh + 1e-16)
    x1 = jnp.maximum(0.25 * acc + skip1[...], 0.0)
    q2[...] = _mm(x1, wq[...], bq[...])
    k2 = _mm(x1, wk[...], bk[...])
    v2 = _mm(x1, wv[...], bv[...])
    kv2[...] = jnp.concatenate([k2, v2], axis=1)
    skip2[...] = _mm(x1, wsk[...], bsk[...])


def _tcc_body(pa, pb, skip2, wo, bo, out):
    num = pa[...][:, :32] + pb[...][:, :32]
    den = jnp.broadcast_to(pa[...][:, 32:33] + pb[...][:, 32:33], (_BLK, MD))
    x2 = jnp.maximum(num / (den + 1e-16) + skip2[...], 0.0)
    out[...] = _mm(x2, wo[...], bo[...])


_MESH = plsc.VectorSubcoreMesh(core_axis_name="c", subcore_axis_name="s")
_SC_PARAMS = pltpu.CompilerParams(use_tc_tiling_on_sc=False,
                                  needs_layout_passes=False)


def _sc_edge_pass(nheads, qdim, row_words, ce,
                  ed_h, q_h, kv_h, out_h,
                  ed_c0, ed_c1, src_c0, src_c1, dst_c0, dst_c1,
                  qv0, qv1, kv0, kv1, con_v, acc_sh,
                  si0, si1, sq0, sq1, skv0, skv1, ss):
    cid = lax.axis_index("c")
    sid = lax.axis_index("s")
    wid = cid * NS + sid
    nchunk = EPW // ce

    ed_c = (ed_c0, ed_c1)
    src_c = (src_c0, src_c1)
    dst_c = (dst_c0, dst_c1)
    qv = (qv0, qv1)
    kv = (kv0, kv1)
    si = (si0, si1)
    sq = (sq0, sq1)
    skv = (skv0, skv1)

    iota16 = lax.iota(jnp.int32, 16)
    z16 = jnp.zeros((16,), _F32)
    nb = row_words // 16

    def zrow(r, c):
        for i in range(nb):
            con_v[r, pl.ds(16 * i, 16)] = z16
        return c
    lax.fori_loop(0, ce, zrow, 0)

    row0 = sid * RPT
    ncopy = jnp.where(sid < 15, RPT // ce, RPT_LAST // ce)

    def init(i, c):
        pltpu.sync_copy(con_v, acc_sh.at[pl.ds(row0 + i * ce, ce)])
        return c
    lax.fori_loop(0, ncopy, init, 0)
    plsc.subcore_barrier()

    ebase = wid * EPW

    def fetch_idx(chunk_id, p):
        b = pl.multiple_of(2 * (ebase + chunk_id * ce), 8)
        pltpu.async_copy(ed_h.at[pl.ds(b, 2 * ce)], ed_c[p], si[p])

    def wait_idx(p):
        b0 = pl.multiple_of(2 * ebase, 8)
        pltpu.make_async_copy(ed_h.at[pl.ds(b0, 2 * ce)], ed_c[p], si[p]).wait()

    def deinterleave(p):
        off = 0
        while off < ce:
            o = min(off, ce - 16)
            lanes = 2 * (o + iota16)
            src_c[p][pl.ds(o, 16)] = plsc.load_gather(ed_c[p], [lanes])
            dst_c[p][pl.ds(o, 16)] = plsc.load_gather(ed_c[p], [lanes + 1])
            off += 16

    def issue_gathers(p):
        pltpu.async_copy(q_h.at[dst_c[p]], qv[p], sq[p])
        pltpu.async_copy(kv_h.at[src_c[p]], kv[p], skv[p])

    def wait_gathers(p):
        pltpu.make_async_copy(q_h.at[dst_c[p]], qv[p], sq[p]).wait()
        pltpu.make_async_copy(kv_h.at[src_c[p]], kv[p], skv[p]).wait()

    def wait_scatter(p):
        pltpu.make_async_copy(con_v, acc_sh.at[dst_c[p]], ss).wait()

    def compute_chunk(p):
        qr, kr = qv[p], kv[p]

        @plsc.parallel_loop(0, ce, 1, unroll=4)
        def edge(e):
            dv = jnp.zeros((16,), _F32)
            for h in range(nheads):
                qa = qr[e, pl.ds(32 * h, 16)]
                qb = qr[e, pl.ds(32 * h + 16, 16)]
                ka = kr[e, pl.ds(32 * h, 16)]
                kb = kr[e, pl.ds(32 * h + 16, 16)]
                alpha = jnp.sum(qa * ka + qb * kb)
                ex = jnp.exp(jnp.full((16,), alpha, _F32))
                va = kr[e, pl.ds(qdim + 32 * h, 16)]
                vb = kr[e, pl.ds(qdim + 32 * h + 16, 16)]
                con_v[e, pl.ds(32 * h, 16)] = ex * va
                con_v[e, pl.ds(32 * h + 16, 16)] = ex * vb
                dv = jnp.where(iota16 == h, ex, dv)
            con_v[e, pl.ds(qdim, 16)] = dv

    fetch_idx(0, 0)
    wait_idx(0)
    deinterleave(0)
    issue_gathers(0)
    fetch_idx(1, 1)

    def pair(t, c):
        for p in (0, 1):
            jj = 2 * t + p
            q = 1 - p
            wait_gathers(p)
            if p == 0:
                @pl.when(t > 0)
                def _():
                    wait_scatter(p)
            else:
                wait_scatter(p)
            wait_idx(q)
            deinterleave(q)
            issue_gathers(q)
            nxt = jnp.minimum(jj + 2, nchunk - 1)
            fetch_idx(nxt, p)
            compute_chunk(p)
            pltpu.async_copy(con_v, acc_sh.at[dst_c[p]], ss, add=True)
        return c

    lax.fori_loop(0, nchunk // 2, pair, 0)

    wait_scatter(0)
    wait_gathers(0)
    wait_idx(1)
    plsc.subcore_barrier()

    @pl.when(sid < 15)
    def _():
        pltpu.sync_copy(acc_sh.at[pl.ds(row0, RPT)],
                        out_h.at[cid, pl.ds(row0, RPT)])

    @pl.when(sid == 15)
    def _():
        pltpu.sync_copy(acc_sh.at[pl.ds(15 * RPT, RPT_LAST)],
                        out_h.at[cid, pl.ds(15 * RPT, RPT_LAST)])


@functools.partial(
    pl.kernel,
    out_type=jax.ShapeDtypeStruct((NC, N, ROW1), _F32),
    mesh=_MESH,
    scratch_types=[
        pltpu.VMEM((2 * CE1,), jnp.int32), pltpu.VMEM((2 * CE1,), jnp.int32),
        pltpu.VMEM((CE1,), jnp.int32), pltpu.VMEM((CE1,), jnp.int32),
        pltpu.VMEM((CE1,), jnp.int32), pltpu.VMEM((CE1,), jnp.int32),
        pltpu.VMEM((CE1, 4 * MD), _F32), pltpu.VMEM((CE1, 4 * MD), _F32),
        pltpu.VMEM((CE1, 8 * MD), _F32), pltpu.VMEM((CE1, 8 * MD), _F32),
        pltpu.VMEM((CE1, ROW1), _F32),
        pltpu.VMEM_SHARED((N, ROW1), _F32),
        pltpu.SemaphoreType.DMA, pltpu.SemaphoreType.DMA,
        pltpu.SemaphoreType.DMA, pltpu.SemaphoreType.DMA,
        pltpu.SemaphoreType.DMA, pltpu.SemaphoreType.DMA,
        pltpu.SemaphoreType.DMA,
    ],
    compiler_params=_SC_PARAMS,
)
def _sc_layer1(*refs):
    _sc_edge_pass(4, 4 * MD, ROW1, CE1, *refs)


@functools.partial(
    pl.kernel,
    out_type=jax.ShapeDtypeStruct((NC, N, ROW2), _F32),
    mesh=_MESH,
    scratch_types=[
        pltpu.VMEM((2 * CE2,), jnp.int32), pltpu.VMEM((2 * CE2,), jnp.int32),
        pltpu.VMEM((CE2,), jnp.int32), pltpu.VMEM((CE2,), jnp.int32),
        pltpu.VMEM((CE2,), jnp.int32), pltpu.VMEM((CE2,), jnp.int32),
        pltpu.VMEM((CE2, MD), _F32), pltpu.VMEM((CE2, MD), _F32),
        pltpu.VMEM((CE2, 2 * MD), _F32), pltpu.VMEM((CE2, 2 * MD), _F32),
        pltpu.VMEM((CE2, ROW2), _F32),
        pltpu.VMEM_SHARED((N, ROW2), _F32),
        pltpu.SemaphoreType.DMA, pltpu.SemaphoreType.DMA,
        pltpu.SemaphoreType.DMA, pltpu.SemaphoreType.DMA,
        pltpu.SemaphoreType.DMA, pltpu.SemaphoreType.DMA,
        pltpu.SemaphoreType.DMA,
    ],
    compiler_params=_SC_PARAMS,
)
def _sc_layer2(*refs):
    _sc_edge_pass(1, MD, ROW2, CE2, *refs)


def kernel(edge_df, node_features, edge_mask, node_mask, Wt, bt, Wn, bn,
           Wq1, bq1, Wk1, bk1, Wv1, bv1, Wsk1, bsk1,
           Wq2, bq2, Wk2, bk2, Wv2, bv2, Wsk2, bsk2,
           Wih, bih, Whh, bhh, Wout, bout, node_memory):
    inv = 1.0 / jnp.sqrt(jnp.asarray(MD, _F32))
    te = Wt[:, 0] + bt

    def fold(W, b, scale=None):
        wa = W[:, :MD].T
        be = b + W[:, MD:] @ te
        if scale is not None:
            wa = wa * scale
            be = be * scale
        return wa, be.reshape(1, -1)

    wq1, bq1e = fold(Wq1, bq1, inv)
    wk1, bk1e = fold(Wk1, bk1)
    wv1, bv1e = fold(Wv1, bv1)
    wsk1, bsk1e = fold(Wsk1, bsk1)

    ed_view = edge_df.reshape(N, IN_DIM)

    q1, kv1, skip1, edi = pl.pallas_call(
        _tca_body,
        grid=(_GRID,),
        in_specs=[
            _row_spec(IN_DIM), _row_spec(IN_DIM),
            _full_spec((IN_DIM, MD)), _full_spec((1, MD)),
            _full_spec((MD, 4 * MD)), _full_spec((1, 4 * MD)),
            _full_spec((MD, 4 * MD)), _full_spec((1, 4 * MD)),
            _full_spec((MD, 4 * MD)), _full_spec((1, 4 * MD)),
            _full_spec((MD, MD)), _full_spec((1, MD)),
        ],
        out_specs=(
            _row_spec(4 * MD), _row_spec(8 * MD), _row_spec(MD),
            _row_spec(IN_DIM),
        ),
        out_shape=(
            jax.ShapeDtypeStruct((N, 4 * MD), _F32),
            jax.ShapeDtypeStruct((N, 8 * MD), _F32),
            jax.ShapeDtypeStruct((N, MD), _F32),
            jax.ShapeDtypeStruct((N, IN_DIM), jnp.int32),
        ),
    )(node_features, ed_view, Wn.T, bn.reshape(1, -1),
      wq1, bq1e, wk1, bk1e, wv1, bv1e, wsk1, bsk1e)

    ed_flat = edi.reshape(2 * E)

    part1 = _sc_layer1(ed_flat, q1, kv1)

    q2, kv2, skip2 = pl.pallas_call(
        _tcb_body,
        grid=(_GRID,),
        in_specs=[
            _row_spec(ROW1), _row_spec(ROW1), _row_spec(MD),
            _full_spec((MD, MD)), _full_spec((1, MD)),
            _full_spec((MD, MD)), _full_spec((1, MD)),
            _full_spec((MD, MD)), _full_spec((1, MD)),
            _full_spec((MD, MD)), _full_spec((1, MD)),
        ],
        out_specs=(_row_spec(MD), _row_spec(2 * MD), _row_spec(MD)),
        out_shape=(
            jax.ShapeDtypeStruct((N, MD), _F32),
            jax.ShapeDtypeStruct((N, 2 * MD), _F32),
            jax.ShapeDtypeStruct((N, MD), _F32),
        ),
    )(part1[0], part1[1], skip1,
      Wq2.T * inv, (bq2 * inv).reshape(1, -1),
      Wk2.T, bk2.reshape(1, -1), Wv2.T, bv2.reshape(1, -1),
      Wsk2.T, bsk2.reshape(1, -1))

    part2 = _sc_layer2(ed_flat, q2, kv2)

    output = pl.pallas_call(
        _tcc_body,
        grid=(_GRID,),
        in_specs=[
            _row_spec(ROW2), _row_spec(ROW2), _row_spec(MD),
            _full_spec((MD, MD)), _full_spec((1, MD)),
        ],
        out_specs=_row_spec(MD),
        out_shape=jax.ShapeDtypeStruct((N, MD), _F32),
    )(part2[0], part2[1], skip2, Wout.T, bout.reshape(1, -1))

    return output


# speedup vs baseline: 2.1229x; 1.0006x over previous
import functools

import jax
import jax.numpy as jnp
from jax import lax
from jax.experimental import pallas as pl
from jax.experimental.pallas import tpu as pltpu
from jax.experimental.pallas import tpu_sc as plsc

N = 10000
E = 640000
IN_DIM = 128
MD = 32
MD4 = 8

NC = 2
NS = 16
NW = NC * NS
EPW = E // NW
CE1 = 40
CE2 = 80
RPT = 640
RPT_LAST = N - 15 * RPT

ROW1 = 144
ROW2 = 48

_F32 = jnp.float32
_BLK = 2000
_GRID = N // _BLK


def _mm(x, w, b):
    return jnp.dot(x, w, preferred_element_type=_F32) + b


def _row_spec(cols):
    return pl.BlockSpec((_BLK, cols), lambda i: (i, 0))


def _full_spec(shape):
    return pl.BlockSpec(shape, lambda i: tuple(0 for _ in shape))


def _tca_body(nf, ed, wn, bn, wq, bq, wk, bk, wv, bv, wsk, bsk,
              q1, kv1, skip1, edi):
    ne = _mm(nf[...], wn[...], bn[...])
    q1[...] = _mm(ne, wq[...], bq[...])
    k1 = _mm(ne, wk[...], bk[...])
    v1 = _mm(ne, wv[...], bv[...])
    kv1[...] = jnp.concatenate([k1, v1], axis=1)
    skip1[...] = _mm(ne, wsk[...], bsk[...])
    edi[...] = jnp.clip(ed[...].astype(jnp.int32), 0, N - 1)


def _tcb_body(pa, pb, skip1, wq, bq, wk, bk, wv, bv, wsk, bsk,
              q2, kv2, skip2):
    num = pa[...][:, :128] + pb[...][:, :128]
    den = pa[...][:, 128:132] + pb[...][:, 128:132]
    acc = jnp.zeros((_BLK, MD), _F32)
    for h in range(4):
        nh = num[:, 32 * h:32 * h + 32]
        dh = jnp.broadcast_to(den[:, h:h + 1], (_BLK, MD))
        acc = acc + nh / (dh + 1e-16)
    x1 = jnp.maximum(0.25 * acc + skip1[...], 0.0)
    q2[...] = _mm(x1, wq[...], bq[...])
    k2 = _mm(x1, wk[...], bk[...])
    v2 = _mm(x1, wv[...], bv[...])
    kv2[...] = jnp.concatenate([k2, v2], axis=1)
    skip2[...] = _mm(x1, wsk[...], bsk[...])


def _tcc_body(pa, pb, skip2, wo, bo, out):
    num = pa[...][:, :32] + pb[...][:, :32]
    den = jnp.broadcast_to(pa[...][:, 32:33] + pb[...][:, 32:33], (_BLK, MD))
    x2 = jnp.maximum(num / (den + 1e-16) + skip2[...], 0.0)
    out[...] = _mm(x2, wo[...], bo[...])


_MESH = plsc.VectorSubcoreMesh(core_axis_name="c", subcore_axis_name="s")
_SC_PARAMS = pltpu.CompilerParams(use_tc_tiling_on_sc=False,
                                  needs_layout_passes=False)


def _sc_edge_pass(nheads, qdim, row_words, ce,
                  ed_h, q_h, kv_h, out_h,
                  ed_c0, ed_c1, src_c0, src_c1, dst_c0, dst_c1,
                  qv0, qv1, kv0, kv1, con_v, acc_sh,
                  si0, si1, sq0, sq1, skv0, skv1, ss):
    cid = lax.axis_index("c")
    sid = lax.axis_index("s")
    wid = cid * NS + sid
    nchunk = EPW // ce

    ed_c = (ed_c0, ed_c1)
    src_c = (src_c0, src_c1)
    dst_c = (dst_c0, dst_c1)
    qv = (qv0, qv1)
    kv = (kv0, kv1)
    si = (si0, si1)
    sq = (sq0, sq1)
    skv = (skv0, skv1)

    iota16 = lax.iota(jnp.int32, 16)
    z16 = jnp.zeros((16,), _F32)
    nb = row_words // 16

    def zrow(r, c):
        for i in range(nb):
            con_v[r, pl.ds(16 * i, 16)] = z16
        return c
    lax.fori_loop(0, ce, zrow, 0)

    row0 = sid * RPT
    ncopy = jnp.where(sid < 15, RPT // ce, RPT_LAST // ce)

    def init(i, c):
        pltpu.sync_copy(con_v, acc_sh.at[pl.ds(row0 + i * ce, ce)])
        return c
    lax.fori_loop(0, ncopy, init, 0)
    plsc.subcore_barrier()

    ebase = wid * EPW

    def fetch_idx(chunk_id, p):
        b = pl.multiple_of(2 * (ebase + chunk_id * ce), 8)
        pltpu.async_copy(ed_h.at[pl.ds(b, 2 * ce)], ed_c[p], si[p])

    def wait_idx(p):
        b0 = pl.multiple_of(2 * ebase, 8)
        pltpu.make_async_copy(ed_h.at[pl.ds(b0, 2 * ce)], ed_c[p], si[p]).wait()

    def deinterleave(p):
        off = 0
        while off < ce:
            o = min(off, ce - 16)
            lanes = 2 * (o + iota16)
            src_c[p][pl.ds(o, 16)] = plsc.load_gather(ed_c[p], [lanes])
            dst_c[p][pl.ds(o, 16)] = plsc.load_gather(ed_c[p], [lanes + 1])
            off += 16

    def issue_gathers(p):
        pltpu.async_copy(q_h.at[dst_c[p]], qv[p], sq[p])
        pltpu.async_copy(kv_h.at[src_c[p]], kv[p], skv[p])

    def wait_gathers(p):
        pltpu.make_async_copy(q_h.at[dst_c[p]], qv[p], sq[p]).wait()
        pltpu.make_async_copy(kv_h.at[src_c[p]], kv[p], skv[p]).wait()

    def wait_scatter(p):
        pltpu.make_async_copy(con_v, acc_sh.at[dst_c[p]], ss).wait()

    def compute_chunk(p):
        qr, kr = qv[p], kv[p]

        @plsc.parallel_loop(0, ce, 1, unroll=4)
        def edge(e):
            dv = jnp.zeros((16,), _F32)
            for h in range(nheads):
                qa = qr[e, pl.ds(32 * h, 16)]
                qb = qr[e, pl.ds(32 * h + 16, 16)]
                ka = kr[e, pl.ds(32 * h, 16)]
                kb = kr[e, pl.ds(32 * h + 16, 16)]
                alpha = jnp.sum(qa * ka + qb * kb)
                ex = jnp.exp(jnp.full((16,), alpha, _F32))
                va = kr[e, pl.ds(qdim + 32 * h, 16)]
                vb = kr[e, pl.ds(qdim + 32 * h + 16, 16)]
                con_v[e, pl.ds(32 * h, 16)] = ex * va
                con_v[e, pl.ds(32 * h + 16, 16)] = ex * vb
                dv = jnp.where(iota16 == h, ex, dv)
            con_v[e, pl.ds(qdim, 16)] = dv

    fetch_idx(0, 0)
    wait_idx(0)
    deinterleave(0)
    issue_gathers(0)
    fetch_idx(1, 1)

    def pair(t, c):
        for p in (0, 1):
            jj = 2 * t + p
            q = 1 - p
            wait_gathers(p)
            if p == 0:
                @pl.when(t > 0)
                def _():
                    wait_scatter(p)
            else:
                wait_scatter(p)
            wait_idx(q)
            deinterleave(q)
            issue_gathers(q)
            nxt = jnp.minimum(jj + 2, nchunk - 1)
            fetch_idx(nxt, p)
            compute_chunk(p)
            pltpu.async_copy(con_v, acc_sh.at[dst_c[p]], ss, add=True)
        return c

    lax.fori_loop(0, nchunk // 2, pair, 0)

    wait_scatter(0)
    wait_gathers(0)
    wait_idx(1)
    plsc.subcore_barrier()

    @pl.when(sid < 15)
    def _():
        pltpu.sync_copy(acc_sh.at[pl.ds(row0, RPT)],
                        out_h.at[cid, pl.ds(row0, RPT)])

    @pl.when(sid == 15)
    def _():
        pltpu.sync_copy(acc_sh.at[pl.ds(15 * RPT, RPT_LAST)],
                        out_h.at[cid, pl.ds(15 * RPT, RPT_LAST)])


@functools.partial(
    pl.kernel,
    out_type=jax.ShapeDtypeStruct((NC, N, ROW1), _F32),
    mesh=_MESH,
    scratch_types=[
        pltpu.VMEM((2 * CE1,), jnp.int32), pltpu.VMEM((2 * CE1,), jnp.int32),
        pltpu.VMEM((CE1,), jnp.int32), pltpu.VMEM((CE1,), jnp.int32),
        pltpu.VMEM((CE1,), jnp.int32), pltpu.VMEM((CE1,), jnp.int32),
        pltpu.VMEM((CE1, 4 * MD), _F32), pltpu.VMEM((CE1, 4 * MD), _F32),
        pltpu.VMEM((CE1, 8 * MD), _F32), pltpu.VMEM((CE1, 8 * MD), _F32),
        pltpu.VMEM((CE1, ROW1), _F32),
        pltpu.VMEM_SHARED((N, ROW1), _F32),
        pltpu.SemaphoreType.DMA, pltpu.SemaphoreType.DMA,
        pltpu.SemaphoreType.DMA, pltpu.SemaphoreType.DMA,
        pltpu.SemaphoreType.DMA, pltpu.SemaphoreType.DMA,
        pltpu.SemaphoreType.DMA,
    ],
    compiler_params=_SC_PARAMS,
)
def _sc_layer1(*refs):
    _sc_edge_pass(4, 4 * MD, ROW1, CE1, *refs)


@functools.partial(
    pl.kernel,
    out_type=jax.ShapeDtypeStruct((NC, N, ROW2), _F32),
    mesh=_MESH,
    scratch_types=[
        pltpu.VMEM((2 * CE2,), jnp.int32), pltpu.VMEM((2 * CE2,), jnp.int32),
        pltpu.VMEM((CE2,), jnp.int32), pltpu.VMEM((CE2,), jnp.int32),
        pltpu.VMEM((CE2,), jnp.int32), pltpu.VMEM((CE2,), jnp.int32),
        pltpu.VMEM((CE2, MD), _F32), pltpu.VMEM((CE2, MD), _F32),
        pltpu.VMEM((CE2, 2 * MD), _F32), pltpu.VMEM((CE2, 2 * MD), _F32),
        pltpu.VMEM((CE2, ROW2), _F32),
        pltpu.VMEM_SHARED((N, ROW2), _F32),
        pltpu.SemaphoreType.DMA, pltpu.SemaphoreType.DMA,
        pltpu.SemaphoreType.DMA, pltpu.SemaphoreType.DMA,
        pltpu.SemaphoreType.DMA, pltpu.SemaphoreType.DMA,
        pltpu.SemaphoreType.DMA,
    ],
    compiler_params=_SC_PARAMS,
)
def _sc_layer2(*refs):
    _sc_edge_pass(1, MD, ROW2, CE2, *refs)


def kernel(edge_df, node_features, edge_mask, node_mask, Wt, bt, Wn, bn,
           Wq1, bq1, Wk1, bk1, Wv1, bv1, Wsk1, bsk1,
           Wq2, bq2, Wk2, bk2, Wv2, bv2, Wsk2, bsk2,
           Wih, bih, Whh, bhh, Wout, bout, node_memory):
    inv = 1.0 / jnp.sqrt(jnp.asarray(MD, _F32))
    te = Wt[:, 0] + bt

    def fold(W, b, scale=None):
        wa = W[:, :MD].T
        be = b + W[:, MD:] @ te
        if scale is not None:
            wa = wa * scale
            be = be * scale
        return wa, be.reshape(1, -1)

    wq1, bq1e = fold(Wq1, bq1, inv)
    wk1, bk1e = fold(Wk1, bk1)
    wv1, bv1e = fold(Wv1, bv1)
    wsk1, bsk1e = fold(Wsk1, bsk1)

    ed_view = edge_df.reshape(N, IN_DIM)

    q1, kv1, skip1, edi = pl.pallas_call(
        _tca_body,
        grid=(_GRID,),
        in_specs=[
            _row_spec(IN_DIM), _row_spec(IN_DIM),
            _full_spec((IN_DIM, MD)), _full_spec((1, MD)),
            _full_spec((MD, 4 * MD)), _full_spec((1, 4 * MD)),
            _full_spec((MD, 4 * MD)), _full_spec((1, 4 * MD)),
            _full_spec((MD, 4 * MD)), _full_spec((1, 4 * MD)),
            _full_spec((MD, MD)), _full_spec((1, MD)),
        ],
        out_specs=(
            _row_spec(4 * MD), _row_spec(8 * MD), _row_spec(MD),
            _row_spec(IN_DIM),
        ),
        out_shape=(
            jax.ShapeDtypeStruct((N, 4 * MD), _F32),
            jax.ShapeDtypeStruct((N, 8 * MD), _F32),
            jax.ShapeDtypeStruct((N, MD), _F32),
            jax.ShapeDtypeStruct((N, IN_DIM), jnp.int32),
        ),
    )(node_features, ed_view, Wn.T, bn.reshape(1, -1),
      wq1, bq1e, wk1, bk1e, wv1, bv1e, wsk1, bsk1e)

    ed_flat = edi.reshape(2 * E)

    part1 = _sc_layer1(ed_flat, q1, kv1)

    q2, kv2, skip2 = pl.pallas_call(
        _tcb_body,
        grid=(_GRID,),
        in_specs=[
            _row_spec(ROW1), _row_spec(ROW1), _row_spec(MD),
            _full_spec((MD, MD)), _full_spec((1, MD)),
            _full_spec((MD, MD)), _full_spec((1, MD)),
            _full_spec((MD, MD)), _full_spec((1, MD)),
            _full_spec((MD, MD)), _full_spec((1, MD)),
        ],
        out_specs=(_row_spec(MD), _row_spec(2 * MD), _row_spec(MD)),
        out_shape=(
            jax.ShapeDtypeStruct((N, MD), _F32),
            jax.ShapeDtypeStruct((N, 2 * MD), _F32),
            jax.ShapeDtypeStruct((N, MD), _F32),
        ),
    )(part1[0], part1[1], skip1,
      Wq2.T * inv, (bq2 * inv).reshape(1, -1),
      Wk2.T, bk2.reshape(1, -1), Wv2.T, bv2.reshape(1, -1),
      Wsk2.T, bsk2.reshape(1, -1))

    part2 = _sc_layer2(ed_flat, q2, kv2)

    output = pl.pallas_call(
        _tcc_body,
        grid=(_GRID,),
        in_specs=[
            _row_spec(ROW2), _row_spec(ROW2), _row_spec(MD),
            _full_spec((MD, MD)), _full_spec((1, MD)),
        ],
        out_specs=_row_spec(MD),
        out_shape=jax.ShapeDtypeStruct((N, MD), _F32),
    )(part2[0], part2[1], skip2, Wout.T, bout.reshape(1, -1))

    return output
